# baseline (speedup 1.0000x reference)
import jax
import jax.numpy as jnp
from jax.experimental import pallas as pl
from jax.experimental.pallas import tpu as pltpu

_N = 8192
_D = 256
_BM = 512
_NBLK = _N // _BM
_HALF = _N // 2


def _gcn_kernel(x_ref, adjl_ref, adjr_ref, w1_ref, b1_ref, w2_ref, b2_ref,
                out_ref, h_ref):
    i = pl.program_id(0)

    @pl.when(i == 0)
    def _compute_hidden():
        xb = x_ref[...].astype(jnp.bfloat16)
        w1b = w1_ref[...].astype(jnp.bfloat16)
        h1 = jax.lax.dot_general(
            xb, w1b, (((1,), (1,)), ((), ())),
            preferred_element_type=jnp.float32)
        h1 = jnp.maximum(h1 + b1_ref[...], 0.0)
        w2b = w2_ref[...].astype(jnp.bfloat16)
        h2 = jax.lax.dot_general(
            h1.astype(jnp.bfloat16), w2b, (((1,), (1,)), ((), ())),
            preferred_element_type=jnp.float32)
        h_ref[...] = (h2 + b2_ref[...]).astype(jnp.bfloat16)

    acc = jnp.dot(
        adjl_ref[...].astype(jnp.bfloat16), h_ref[:_HALF, :],
        preferred_element_type=jnp.float32)
    acc += jnp.dot(
        adjr_ref[...].astype(jnp.bfloat16), h_ref[_HALF:, :],
        preferred_element_type=jnp.float32)
    out_ref[...] = acc


def kernel(x, adj, W1, b1, W2, b2):
    b1r = b1.reshape(1, _D)
    b2r = b2.reshape(1, _D)
    return pl.pallas_call(
        _gcn_kernel,
        grid=(_NBLK,),
        in_specs=[
            pl.BlockSpec((_N, _D), lambda i: (0, 0)),
            pl.BlockSpec((_BM, _HALF), lambda i: (i, 0)),
            pl.BlockSpec((_BM, _HALF), lambda i: (i, 1)),
            pl.BlockSpec((_D, _D), lambda i: (0, 0)),
            pl.BlockSpec((1, _D), lambda i: (0, 0)),
            pl.BlockSpec((_D, _D), lambda i: (0, 0)),
            pl.BlockSpec((1, _D), lambda i: (0, 0)),
        ],
        out_specs=pl.BlockSpec((_BM, _D), lambda i: (i, 0)),
        out_shape=jax.ShapeDtypeStruct((_N, _D), jnp.float32),
        scratch_shapes=[pltpu.VMEM((_N, _D), jnp.bfloat16)],
    )(x, adj, adj, W1, b1r, W2, b2r)


# speedup vs baseline: 1.0313x; 1.0313x over previous
import jax
import jax.numpy as jnp
from jax.experimental import pallas as pl
from jax.experimental.pallas import tpu as pltpu

_BM = 512


def _gcn_kernel(x_ref, adj_ref, w1_ref, b1_ref, w2_ref, b2_ref, out_ref, h_ref):
    i = pl.program_id(0)

    @pl.when(i == 0)
    def _compute_hidden():
        xb = x_ref[...].astype(jnp.bfloat16)
        w1b = w1_ref[...].astype(jnp.bfloat16)
        h1 = jax.lax.dot_general(
            xb, w1b, (((1,), (1,)), ((), ())),
            preferred_element_type=jnp.float32)
        h1 = jnp.maximum(h1 + b1_ref[...], 0.0)
        w2b = w2_ref[...].astype(jnp.bfloat16)
        h2 = jax.lax.dot_general(
            h1.astype(jnp.bfloat16), w2b, (((1,), (1,)), ((), ())),
            preferred_element_type=jnp.float32)
        h_ref[...] = (h2 + b2_ref[...]).astype(jnp.bfloat16)

    out_ref[...] = jnp.dot(
        adj_ref[...].astype(jnp.bfloat16), h_ref[...],
        preferred_element_type=jnp.float32)


def kernel(x, adj, W1, b1, W2, b2):
    n, d_in = x.shape
    d_out = W2.shape[0]
    b1r = b1.reshape(1, d_out)
    b2r = b2.reshape(1, d_out)
    return pl.pallas_call(
        _gcn_kernel,
        grid=(n // _BM,),
        in_specs=[
            pl.BlockSpec((n, d_in), lambda i: (0, 0)),
            pl.BlockSpec((_BM, n), lambda i: (i, 0)),
            pl.BlockSpec((W1.shape[0], d_in), lambda i: (0, 0)),
            pl.BlockSpec((1, d_out), lambda i: (0, 0)),
            pl.BlockSpec((d_out, d_out), lambda i: (0, 0)),
            pl.BlockSpec((1, d_out), lambda i: (0, 0)),
        ],
        out_specs=pl.BlockSpec((_BM, d_out), lambda i: (i, 0)),
        out_shape=jax.ShapeDtypeStruct((n, d_out), jnp.float32),
        scratch_shapes=[pltpu.VMEM((n, d_out), jnp.bfloat16)],
    )(x, adj, W1, b1r, W2, b2r)
